# baseline (speedup 1.0000x reference)
import functools

import jax
import jax.numpy as jnp
from jax import lax
from jax.experimental import pallas as pl
from jax.experimental.pallas import tpu as pltpu
from jax.experimental.pallas import tpu_sc as plsc

N_NODES = 10000
N_PAD = 10240
D = 128
NUM_CLASSES_PAD = 32
NUM_GRAPHS = 64

NC = 2
NS = 16
NW = NC * NS
CHUNK = 48
CHUNKS_PER_W = 212
ZBLK = 32
E_PER_W = CHUNK * CHUNKS_PER_W
E_PAD = NW * E_PER_W
ROWS_PER_TILE = N_PAD // NS

BN = 2000

@functools.cache
def _sc_mesh():
    return plsc.VectorSubcoreMesh(
        core_axis_name="c", subcore_axis_name="s", num_cores=NC, num_subcores=NS
    )


def _worker_id():
    return lax.axis_index("s") * NC + lax.axis_index("c")


NBUF_DEG = 4


def _sc_degree_body(dst_hbm, deg_out, dst_v, ones_v, zero_v,
                    ds0, ds1, ds2, ds3, deg_sh):
    cid = lax.axis_index("c")
    sid = lax.axis_index("s")
    wid = _worker_id()
    sems = (ds0, ds1, ds2, ds3)

    pltpu.sync_copy(dst_hbm.at[wid], dst_v)

    for i in range(CHUNK // 16):
        ones_v[pl.ds(i * 16, 16)] = jnp.ones((16,), jnp.float32)
        zero_v[pl.ds(i * 16, 16)] = jnp.zeros((16,), jnp.float32)

    row0 = sid * ROWS_PER_TILE
    for j in range(ROWS_PER_TILE // ZBLK):
        pltpu.sync_copy(
            zero_v.at[pl.ds(0, ZBLK)], deg_sh.at[pl.ds(row0 + j * ZBLK, ZBLK)]
        )
    plsc.subcore_barrier()

    def scat(ci, b):
        return pltpu.async_copy(ones_v, deg_sh.at[dst_v.at[ci]], sems[b], add=True)

    def pipe_body(j, carry):
        for b in range(NBUF_DEG):
            ci = j * NBUF_DEG + b

            @pl.when(ci >= NBUF_DEG)
            def _wait_prev():
                pltpu.make_async_copy(
                    ones_v, deg_sh.at[dst_v.at[ci - NBUF_DEG]], sems[b]
                ).wait()

            scat(ci, b)
        return carry

    lax.fori_loop(0, CHUNKS_PER_W // NBUF_DEG, pipe_body, 0)
    for b in range(NBUF_DEG):
        ci = CHUNKS_PER_W - NBUF_DEG + b
        pltpu.make_async_copy(ones_v, deg_sh.at[dst_v.at[ci]], sems[b % NBUF_DEG]).wait()

    plsc.subcore_barrier()
    pltpu.sync_copy(
        deg_sh.at[pl.ds(row0, ROWS_PER_TILE)],
        deg_out.at[cid, pl.ds(row0, ROWS_PER_TILE)],
    )


@functools.cache
def _sc_degree_kernel():
    return pl.kernel(
        _sc_degree_body,
        out_type=jax.ShapeDtypeStruct((NC, N_PAD), jnp.float32),
        mesh=_sc_mesh(),
        scratch_types=[
            pltpu.VMEM((CHUNKS_PER_W, CHUNK), jnp.int32),
            pltpu.VMEM((CHUNK,), jnp.float32),
            pltpu.VMEM((CHUNK,), jnp.float32),
            pltpu.SemaphoreType.DMA,
            pltpu.SemaphoreType.DMA,
            pltpu.SemaphoreType.DMA,
            pltpu.SemaphoreType.DMA,
            pltpu.VMEM_SHARED((N_PAD,), jnp.float32),
        ],
    )


def _sc_degree(dst_p):
    return _sc_degree_kernel()(dst_p)


NBUF = 4


def _sc_propagate_body(idx_hbm, g_hbm, acc_out, idx_v, src_c, dst_c,
                       rows0, rows1, rows2, rows3, gs0, gs1, gs2, gs3,
                       is0, is1, is2, is3, acc_sh):
    cid = lax.axis_index("c")
    sid = lax.axis_index("s")
    wid = _worker_id()
    bufs = (rows0, rows1, rows2, rows3)
    gsems = (gs0, gs1, gs2, gs3)
    isems = (is0, is1, is2, is3)

    def idx_load(ci, slot):
        return pltpu.async_copy(idx_hbm.at[wid, ci], idx_v.at[slot], isems[slot])

    def idx_wait(ci, slot):
        pltpu.make_async_copy(idx_hbm.at[wid, ci], idx_v.at[slot], isems[slot]).wait()

    def decode(slot):
        for i in range(CHUNK // 16):
            w = idx_v[slot, pl.ds(i * 16, 16)]
            src_c[slot, pl.ds(i * 16, 16)] = lax.bitwise_and(w, jnp.int32(0xFFFF))
            dst_c[slot, pl.ds(i * 16, 16)] = lax.shift_right_logical(w, jnp.int32(16))

    for k in range(NBUF):
        idx_load(k, k)

    def zero_body(i, carry):
        r = i // (D // 16)
        col = (i % (D // 16)) * 16
        rows0[r, pl.ds(col, 16)] = jnp.zeros((16,), jnp.float32)
        return carry

    lax.fori_loop(0, ZBLK * (D // 16), zero_body, 0)
    row0 = sid * ROWS_PER_TILE
    for j in range(ROWS_PER_TILE // ZBLK):
        pltpu.sync_copy(
            rows0.at[pl.ds(0, ZBLK)], acc_sh.at[pl.ds(row0 + j * ZBLK, ZBLK)]
        )
    plsc.subcore_barrier()

    def gather(b):
        return pltpu.async_copy(g_hbm.at[src_c.at[b]], bufs[b], gsems[b])

    for k in range(NBUF - 1):
        idx_wait(k, k)
        decode(k)
        gather(k)

    def pipe_body(j, carry):
        for b in range(NBUF):
            ci = j * NBUF + b
            pltpu.make_async_copy(g_hbm.at[src_c.at[b]], bufs[b], gsems[b]).wait()
            sn = (b + NBUF - 1) % NBUF

            @pl.when(ci + NBUF - 1 < CHUNKS_PER_W)
            def _issue_next():
                idx_wait(ci + NBUF - 1, sn)
                decode(sn)

                @pl.when(ci + NBUF < CHUNKS_PER_W)
                def _refill_idx():
                    idx_load(ci + NBUF, b)

                gather(sn)

            pltpu.sync_copy(bufs[b], acc_sh.at[dst_c.at[b]], add=True)

        return carry

    lax.fori_loop(0, CHUNKS_PER_W // NBUF, pipe_body, 0)

    plsc.subcore_barrier()
    pltpu.sync_copy(
        acc_sh.at[pl.ds(row0, ROWS_PER_TILE)],
        acc_out.at[cid, pl.ds(row0, ROWS_PER_TILE)],
    )


@functools.cache
def _sc_propagate_kernel():
    return pl.kernel(
        _sc_propagate_body,
        out_type=jax.ShapeDtypeStruct((NC, N_PAD, D), jnp.float32),
        mesh=_sc_mesh(),
        scratch_types=[
            pltpu.VMEM((NBUF, CHUNK), jnp.int32),
            pltpu.VMEM((NBUF, CHUNK), jnp.int32),
            pltpu.VMEM((NBUF, CHUNK), jnp.int32),
            pltpu.VMEM((CHUNK, D), jnp.float32),
            pltpu.VMEM((CHUNK, D), jnp.float32),
            pltpu.VMEM((CHUNK, D), jnp.float32),
            pltpu.VMEM((CHUNK, D), jnp.float32),
            pltpu.SemaphoreType.DMA,
            pltpu.SemaphoreType.DMA,
            pltpu.SemaphoreType.DMA,
            pltpu.SemaphoreType.DMA,
            pltpu.SemaphoreType.DMA,
            pltpu.SemaphoreType.DMA,
            pltpu.SemaphoreType.DMA,
            pltpu.SemaphoreType.DMA,
            pltpu.VMEM_SHARED((N_PAD, D), jnp.float32),
        ],
    )


def _sc_propagate(idx_p, g):
    return _sc_propagate_kernel()(idx_p, g)


def _tc_prep_body(deg_ref, x_ref, emb_ref, w1_ref, dis_ref, g1_ref):
    deg = deg_ref[:, 0:1] + deg_ref[:, 1:2] + 1.0
    dis = lax.rsqrt(deg)
    dis_ref[...] = dis
    oh = (x_ref[...] == lax.broadcasted_iota(jnp.int32, (BN, NUM_CLASSES_PAD), 1))
    table = jnp.dot(emb_ref[...], w1_ref[...], preferred_element_type=jnp.float32)
    g1_ref[...] = dis * jnp.dot(
        oh.astype(jnp.float32), table, preferred_element_type=jnp.float32
    )


def _tc_prep(deg_t, x2, emb32, w1):
    grid = N_NODES // BN
    return pl.pallas_call(
        _tc_prep_body,
        grid=(grid,),
        in_specs=[
            pl.BlockSpec((BN, NC), lambda i: (i, 0)),
            pl.BlockSpec((BN, 1), lambda i: (i, 0)),
            pl.BlockSpec((NUM_CLASSES_PAD, D), lambda i: (0, 0)),
            pl.BlockSpec((D, D), lambda i: (0, 0)),
        ],
        out_specs=[
            pl.BlockSpec((BN, 1), lambda i: (i, 0)),
            pl.BlockSpec((BN, D), lambda i: (i, 0)),
        ],
        out_shape=[
            jax.ShapeDtypeStruct((N_NODES, 1), jnp.float32),
            jax.ShapeDtypeStruct((N_NODES, D), jnp.float32),
        ],
    )(deg_t, x2, emb32, w1)


def _tc_mid_body(acc_ref, g_ref, dis_ref, b_ref, w_ref, out_ref):
    s = acc_ref[0] + acc_ref[1] + g_ref[...]
    h = jnp.maximum(dis_ref[...] * s + b_ref[...], 0.0)
    out_ref[...] = dis_ref[...] * jnp.dot(
        h, w_ref[...], preferred_element_type=jnp.float32
    )


def _tc_mid(acc, g, dis, b, w_next):
    grid = N_NODES // BN
    return pl.pallas_call(
        _tc_mid_body,
        grid=(grid,),
        in_specs=[
            pl.BlockSpec((NC, BN, D), lambda i: (0, i, 0)),
            pl.BlockSpec((BN, D), lambda i: (i, 0)),
            pl.BlockSpec((BN, 1), lambda i: (i, 0)),
            pl.BlockSpec((1, D), lambda i: (0, 0)),
            pl.BlockSpec((D, D), lambda i: (0, 0)),
        ],
        out_specs=pl.BlockSpec((BN, D), lambda i: (i, 0)),
        out_shape=jax.ShapeDtypeStruct((N_NODES, D), jnp.float32),
    )(acc, g, dis, b, w_next)


def _tc_final_body(acc_ref, g_ref, dis_ref, b_ref, batch_ref, out_ref, sums_ref, cnts_ref):
    i = pl.program_id(0)

    @pl.when(i == 0)
    def _init():
        sums_ref[...] = jnp.zeros_like(sums_ref)
        cnts_ref[...] = jnp.zeros_like(cnts_ref)

    s = acc_ref[0] + acc_ref[1] + g_ref[...]
    h = dis_ref[...] * s + b_ref[...]
    oh_t = (
        lax.broadcasted_iota(jnp.int32, (NUM_GRAPHS, BN), 0) == batch_ref[0]
    ).astype(jnp.float32)
    sums_ref[...] += jnp.dot(oh_t, h, preferred_element_type=jnp.float32)
    cnts_ref[...] += jnp.sum(oh_t, axis=1, keepdims=True)

    @pl.when(i == pl.num_programs(0) - 1)
    def _finish():
        pooled = sums_ref[...] / jnp.maximum(cnts_ref[...], 1.0)
        nrm = jnp.sqrt(jnp.sum(pooled * pooled, axis=1, keepdims=True))
        out_ref[...] = pooled / nrm


def _tc_final(acc, g, dis, b, batch1):
    grid = N_NODES // BN
    return pl.pallas_call(
        _tc_final_body,
        grid=(grid,),
        in_specs=[
            pl.BlockSpec((NC, BN, D), lambda i: (0, i, 0)),
            pl.BlockSpec((BN, D), lambda i: (i, 0)),
            pl.BlockSpec((BN, 1), lambda i: (i, 0)),
            pl.BlockSpec((1, D), lambda i: (0, 0)),
            pl.BlockSpec((1, 1, BN), lambda i: (i, 0, 0)),
        ],
        out_specs=pl.BlockSpec((NUM_GRAPHS, D), lambda i: (0, 0)),
        out_shape=jax.ShapeDtypeStruct((NUM_GRAPHS, D), jnp.float32),
        scratch_shapes=[
            pltpu.VMEM((NUM_GRAPHS, D), jnp.float32),
            pltpu.VMEM((NUM_GRAPHS, 1), jnp.float32),
        ],
    )(acc, g, dis, b, batch1)


def kernel(x, edge_index, batch, emb_table, W1, b1, W2, b2, W3, b3):
    src = edge_index[0]
    dst = edge_index[1]
    e = src.shape[0]
    npad = E_PAD - e
    pi = jnp.arange(npad, dtype=src.dtype)
    src_f = jnp.concatenate([src, pi % N_NODES]).astype(jnp.int32)
    dst_f = jnp.concatenate([dst, N_NODES + pi % (N_PAD - N_NODES)]).astype(jnp.int32)
    dst_p = dst_f.reshape(NW, CHUNKS_PER_W, CHUNK)
    idx_p = (src_f | (dst_f << 16)).reshape(NW, CHUNKS_PER_W, CHUNK)

    emb32 = jnp.pad(emb_table, ((0, NUM_CLASSES_PAD - emb_table.shape[0]), (0, 0)))
    x2 = x.reshape(N_NODES, 1).astype(jnp.int32)
    batch1 = batch.reshape(N_NODES // BN, 1, BN).astype(jnp.int32)

    deg2 = _sc_degree(dst_p)
    deg_t = deg2[:, :N_NODES].T
    dis, g1 = _tc_prep(deg_t, x2, emb32, W1)
    acc1 = _sc_propagate(idx_p, g1)
    g2 = _tc_mid(acc1, g1, dis, b1.reshape(1, D), W2)
    acc2 = _sc_propagate(idx_p, g2)
    g3 = _tc_mid(acc2, g2, dis, b2.reshape(1, D), W3)
    acc3 = _sc_propagate(idx_p, g3)
    return _tc_final(acc3, g3, dis, b3.reshape(1, D), batch1)


# speedup vs baseline: 1.2219x; 1.2219x over previous
import functools

import jax
import jax.numpy as jnp
from jax import lax
from jax.experimental import pallas as pl
from jax.experimental.pallas import tpu as pltpu
from jax.experimental.pallas import tpu_sc as plsc

N_NODES = 10000
N_PAD = 10240
D = 128
NUM_CLASSES_PAD = 32
NUM_GRAPHS = 64

NC = 2
NS = 16
NW = NC * NS
CHUNK = 80
CHUNKS_PER_W = 126
ZBLK = 32
E_PER_W = CHUNK * CHUNKS_PER_W
E_PAD = NW * E_PER_W
ROWS_PER_TILE = N_PAD // NS

BN = 2000

@functools.cache
def _sc_mesh():
    return plsc.VectorSubcoreMesh(
        core_axis_name="c", subcore_axis_name="s", num_cores=NC, num_subcores=NS
    )


def _worker_id():
    return lax.axis_index("s") * NC + lax.axis_index("c")


NBUF_DEG = 3


def _sc_degree_body(dst_hbm, deg_out, dst_v, ones_v, zero_v,
                    ds0, ds1, ds2, ds3, deg_sh):
    cid = lax.axis_index("c")
    sid = lax.axis_index("s")
    wid = _worker_id()
    sems = (ds0, ds1, ds2, ds3)

    pltpu.sync_copy(dst_hbm.at[wid], dst_v)

    for i in range(CHUNK // 16):
        ones_v[pl.ds(i * 16, 16)] = jnp.ones((16,), jnp.float32)
        zero_v[pl.ds(i * 16, 16)] = jnp.zeros((16,), jnp.float32)

    row0 = sid * ROWS_PER_TILE
    for j in range(ROWS_PER_TILE // ZBLK):
        pltpu.sync_copy(
            zero_v.at[pl.ds(0, ZBLK)], deg_sh.at[pl.ds(row0 + j * ZBLK, ZBLK)]
        )
    plsc.subcore_barrier()

    def scat(ci, b):
        return pltpu.async_copy(ones_v, deg_sh.at[dst_v.at[ci]], sems[b], add=True)

    def pipe_body(j, carry):
        for b in range(NBUF_DEG):
            ci = j * NBUF_DEG + b

            @pl.when(ci >= NBUF_DEG)
            def _wait_prev():
                pltpu.make_async_copy(
                    ones_v, deg_sh.at[dst_v.at[ci - NBUF_DEG]], sems[b]
                ).wait()

            scat(ci, b)
        return carry

    lax.fori_loop(0, CHUNKS_PER_W // NBUF_DEG, pipe_body, 0)
    for b in range(NBUF_DEG):
        ci = CHUNKS_PER_W - NBUF_DEG + b
        pltpu.make_async_copy(ones_v, deg_sh.at[dst_v.at[ci]], sems[b % NBUF_DEG]).wait()

    plsc.subcore_barrier()
    pltpu.sync_copy(
        deg_sh.at[pl.ds(row0, ROWS_PER_TILE)],
        deg_out.at[cid, pl.ds(row0, ROWS_PER_TILE)],
    )


@functools.cache
def _sc_degree_kernel():
    return pl.kernel(
        _sc_degree_body,
        out_type=jax.ShapeDtypeStruct((NC, N_PAD), jnp.float32),
        mesh=_sc_mesh(),
        scratch_types=[
            pltpu.VMEM((CHUNKS_PER_W, CHUNK), jnp.int32),
            pltpu.VMEM((CHUNK,), jnp.float32),
            pltpu.VMEM((CHUNK,), jnp.float32),
            pltpu.SemaphoreType.DMA,
            pltpu.SemaphoreType.DMA,
            pltpu.SemaphoreType.DMA,
            pltpu.SemaphoreType.DMA,
            pltpu.VMEM_SHARED((N_PAD,), jnp.float32),
        ],
    )


def _sc_degree(dst_p):
    return _sc_degree_kernel()(dst_p)


NBUF = 3


def _sc_propagate_body(idx_hbm, g_hbm, acc_out, idx_v, src_c, dst_c,
                       rows0, rows1, rows2, gs0, gs1, gs2,
                       is0, is1, is2, acc_sh):
    cid = lax.axis_index("c")
    sid = lax.axis_index("s")
    wid = _worker_id()
    bufs = (rows0, rows1, rows2)
    gsems = (gs0, gs1, gs2)
    isems = (is0, is1, is2)

    def idx_load(ci, slot):
        return pltpu.async_copy(idx_hbm.at[wid, ci], idx_v.at[slot], isems[slot])

    def idx_wait(ci, slot):
        pltpu.make_async_copy(idx_hbm.at[wid, ci], idx_v.at[slot], isems[slot]).wait()

    def decode(slot):
        for i in range(CHUNK // 16):
            w = idx_v[slot, pl.ds(i * 16, 16)]
            src_c[slot, pl.ds(i * 16, 16)] = lax.bitwise_and(w, jnp.int32(0xFFFF))
            dst_c[slot, pl.ds(i * 16, 16)] = lax.shift_right_logical(w, jnp.int32(16))

    for k in range(NBUF):
        idx_load(k, k)

    def zero_body(i, carry):
        r = i // (D // 16)
        col = (i % (D // 16)) * 16
        rows0[r, pl.ds(col, 16)] = jnp.zeros((16,), jnp.float32)
        return carry

    lax.fori_loop(0, ZBLK * (D // 16), zero_body, 0)
    row0 = sid * ROWS_PER_TILE
    for j in range(ROWS_PER_TILE // ZBLK):
        pltpu.sync_copy(
            rows0.at[pl.ds(0, ZBLK)], acc_sh.at[pl.ds(row0 + j * ZBLK, ZBLK)]
        )
    plsc.subcore_barrier()

    def gather(b):
        return pltpu.async_copy(g_hbm.at[src_c.at[b]], bufs[b], gsems[b])

    for k in range(NBUF - 1):
        idx_wait(k, k)
        decode(k)
        gather(k)

    def pipe_body(j, carry):
        for b in range(NBUF):
            ci = j * NBUF + b
            pltpu.make_async_copy(g_hbm.at[src_c.at[b]], bufs[b], gsems[b]).wait()
            sn = (b + NBUF - 1) % NBUF

            @pl.when(ci + NBUF - 1 < CHUNKS_PER_W)
            def _issue_next():
                idx_wait(ci + NBUF - 1, sn)
                decode(sn)

                @pl.when(ci + NBUF < CHUNKS_PER_W)
                def _refill_idx():
                    idx_load(ci + NBUF, b)

                gather(sn)

            pltpu.sync_copy(bufs[b], acc_sh.at[dst_c.at[b]], add=True)

        return carry

    lax.fori_loop(0, CHUNKS_PER_W // NBUF, pipe_body, 0)

    plsc.subcore_barrier()
    pltpu.sync_copy(
        acc_sh.at[pl.ds(row0, ROWS_PER_TILE)],
        acc_out.at[cid, pl.ds(row0, ROWS_PER_TILE)],
    )


@functools.cache
def _sc_propagate_kernel():
    return pl.kernel(
        _sc_propagate_body,
        out_type=jax.ShapeDtypeStruct((NC, N_PAD, D), jnp.float32),
        mesh=_sc_mesh(),
        scratch_types=[
            pltpu.VMEM((NBUF, CHUNK), jnp.int32),
            pltpu.VMEM((NBUF, CHUNK), jnp.int32),
            pltpu.VMEM((NBUF, CHUNK), jnp.int32),
            pltpu.VMEM((CHUNK, D), jnp.float32),
            pltpu.VMEM((CHUNK, D), jnp.float32),
            pltpu.VMEM((CHUNK, D), jnp.float32),
            pltpu.SemaphoreType.DMA,
            pltpu.SemaphoreType.DMA,
            pltpu.SemaphoreType.DMA,
            pltpu.SemaphoreType.DMA,
            pltpu.SemaphoreType.DMA,
            pltpu.SemaphoreType.DMA,
            pltpu.VMEM_SHARED((N_PAD, D), jnp.float32),
        ],
    )


def _sc_propagate(idx_p, g):
    return _sc_propagate_kernel()(idx_p, g)


def _tc_prep_body(deg_ref, x_ref, emb_ref, w1_ref, dis_ref, g1_ref):
    deg = deg_ref[:, 0:1] + deg_ref[:, 1:2] + 1.0
    dis = lax.rsqrt(deg)
    dis_ref[...] = dis
    oh = (x_ref[...] == lax.broadcasted_iota(jnp.int32, (BN, NUM_CLASSES_PAD), 1))
    table = jnp.dot(emb_ref[...], w1_ref[...], preferred_element_type=jnp.float32)
    g1_ref[...] = dis * jnp.dot(
        oh.astype(jnp.float32), table, preferred_element_type=jnp.float32
    )


def _tc_prep(deg_t, x2, emb32, w1):
    grid = N_NODES // BN
    return pl.pallas_call(
        _tc_prep_body,
        grid=(grid,),
        in_specs=[
            pl.BlockSpec((BN, NC), lambda i: (i, 0)),
            pl.BlockSpec((BN, 1), lambda i: (i, 0)),
            pl.BlockSpec((NUM_CLASSES_PAD, D), lambda i: (0, 0)),
            pl.BlockSpec((D, D), lambda i: (0, 0)),
        ],
        out_specs=[
            pl.BlockSpec((BN, 1), lambda i: (i, 0)),
            pl.BlockSpec((BN, D), lambda i: (i, 0)),
        ],
        out_shape=[
            jax.ShapeDtypeStruct((N_NODES, 1), jnp.float32),
            jax.ShapeDtypeStruct((N_NODES, D), jnp.float32),
        ],
    )(deg_t, x2, emb32, w1)


def _tc_mid_body(acc_ref, g_ref, dis_ref, b_ref, w_ref, out_ref):
    s = acc_ref[0] + acc_ref[1] + g_ref[...]
    h = jnp.maximum(dis_ref[...] * s + b_ref[...], 0.0)
    out_ref[...] = dis_ref[...] * jnp.dot(
        h, w_ref[...], preferred_element_type=jnp.float32
    )


def _tc_mid(acc, g, dis, b, w_next):
    grid = N_NODES // BN
    return pl.pallas_call(
        _tc_mid_body,
        grid=(grid,),
        in_specs=[
            pl.BlockSpec((NC, BN, D), lambda i: (0, i, 0)),
            pl.BlockSpec((BN, D), lambda i: (i, 0)),
            pl.BlockSpec((BN, 1), lambda i: (i, 0)),
            pl.BlockSpec((1, D), lambda i: (0, 0)),
            pl.BlockSpec((D, D), lambda i: (0, 0)),
        ],
        out_specs=pl.BlockSpec((BN, D), lambda i: (i, 0)),
        out_shape=jax.ShapeDtypeStruct((N_NODES, D), jnp.float32),
    )(acc, g, dis, b, w_next)


def _tc_final_body(acc_ref, g_ref, dis_ref, b_ref, batch_ref, out_ref, sums_ref, cnts_ref):
    i = pl.program_id(0)

    @pl.when(i == 0)
    def _init():
        sums_ref[...] = jnp.zeros_like(sums_ref)
        cnts_ref[...] = jnp.zeros_like(cnts_ref)

    s = acc_ref[0] + acc_ref[1] + g_ref[...]
    h = dis_ref[...] * s + b_ref[...]
    oh_t = (
        lax.broadcasted_iota(jnp.int32, (NUM_GRAPHS, BN), 0) == batch_ref[0]
    ).astype(jnp.float32)
    sums_ref[...] += jnp.dot(oh_t, h, preferred_element_type=jnp.float32)
    cnts_ref[...] += jnp.sum(oh_t, axis=1, keepdims=True)

    @pl.when(i == pl.num_programs(0) - 1)
    def _finish():
        pooled = sums_ref[...] / jnp.maximum(cnts_ref[...], 1.0)
        nrm = jnp.sqrt(jnp.sum(pooled * pooled, axis=1, keepdims=True))
        out_ref[...] = pooled / nrm


def _tc_final(acc, g, dis, b, batch1):
    grid = N_NODES // BN
    return pl.pallas_call(
        _tc_final_body,
        grid=(grid,),
        in_specs=[
            pl.BlockSpec((NC, BN, D), lambda i: (0, i, 0)),
            pl.BlockSpec((BN, D), lambda i: (i, 0)),
            pl.BlockSpec((BN, 1), lambda i: (i, 0)),
            pl.BlockSpec((1, D), lambda i: (0, 0)),
            pl.BlockSpec((1, 1, BN), lambda i: (i, 0, 0)),
        ],
        out_specs=pl.BlockSpec((NUM_GRAPHS, D), lambda i: (0, 0)),
        out_shape=jax.ShapeDtypeStruct((NUM_GRAPHS, D), jnp.float32),
        scratch_shapes=[
            pltpu.VMEM((NUM_GRAPHS, D), jnp.float32),
            pltpu.VMEM((NUM_GRAPHS, 1), jnp.float32),
        ],
    )(acc, g, dis, b, batch1)


def kernel(x, edge_index, batch, emb_table, W1, b1, W2, b2, W3, b3):
    src = edge_index[0]
    dst = edge_index[1]
    e = src.shape[0]
    npad = E_PAD - e
    pi = jnp.arange(npad, dtype=src.dtype)
    src_f = jnp.concatenate([src, pi % N_NODES]).astype(jnp.int32)
    dst_f = jnp.concatenate([dst, N_NODES + pi % (N_PAD - N_NODES)]).astype(jnp.int32)
    dst_p = dst_f.reshape(NW, CHUNKS_PER_W, CHUNK)
    idx_p = (src_f | (dst_f << 16)).reshape(NW, CHUNKS_PER_W, CHUNK)

    emb32 = jnp.pad(emb_table, ((0, NUM_CLASSES_PAD - emb_table.shape[0]), (0, 0)))
    x2 = x.reshape(N_NODES, 1).astype(jnp.int32)
    batch1 = batch.reshape(N_NODES // BN, 1, BN).astype(jnp.int32)

    deg2 = _sc_degree(dst_p)
    deg_t = deg2[:, :N_NODES].T
    dis, g1 = _tc_prep(deg_t, x2, emb32, W1)
    acc1 = _sc_propagate(idx_p, g1)
    g2 = _tc_mid(acc1, g1, dis, b1.reshape(1, D), W2)
    acc2 = _sc_propagate(idx_p, g2)
    g3 = _tc_mid(acc2, g2, dis, b2.reshape(1, D), W3)
    acc3 = _sc_propagate(idx_p, g3)
    return _tc_final(acc3, g3, dis, b3.reshape(1, D), batch1)


# speedup vs baseline: 1.2577x; 1.0293x over previous
import functools

import jax
import jax.numpy as jnp
from jax import lax
from jax.experimental import pallas as pl
from jax.experimental.pallas import tpu as pltpu
from jax.experimental.pallas import tpu_sc as plsc

N_NODES = 10000
N_PAD = 10240
D = 128
NUM_CLASSES_PAD = 32
NUM_GRAPHS = 64

NC = 2
NS = 16
NW = NC * NS
CHUNK = 96
CHUNKS_PER_W = 105
ZBLK = 32
E_PER_W = CHUNK * CHUNKS_PER_W
E_PAD = NW * E_PER_W
ROWS_PER_TILE = N_PAD // NS

BN = 2000

@functools.cache
def _sc_mesh():
    return plsc.VectorSubcoreMesh(
        core_axis_name="c", subcore_axis_name="s", num_cores=NC, num_subcores=NS
    )


def _worker_id():
    return lax.axis_index("s") * NC + lax.axis_index("c")


NBUF_DEG = 3


def _sc_degree_body(dst_hbm, deg_out, dst_v, ones_v, zero_v,
                    ds0, ds1, ds2, ds3, deg_sh):
    cid = lax.axis_index("c")
    sid = lax.axis_index("s")
    wid = _worker_id()
    sems = (ds0, ds1, ds2, ds3)

    pltpu.sync_copy(dst_hbm.at[wid], dst_v)

    for i in range(CHUNK // 16):
        ones_v[pl.ds(i * 16, 16)] = jnp.ones((16,), jnp.float32)
        zero_v[pl.ds(i * 16, 16)] = jnp.zeros((16,), jnp.float32)

    row0 = sid * ROWS_PER_TILE
    for j in range(ROWS_PER_TILE // ZBLK):
        pltpu.sync_copy(
            zero_v.at[pl.ds(0, ZBLK)], deg_sh.at[pl.ds(row0 + j * ZBLK, ZBLK)]
        )
    plsc.subcore_barrier()

    def scat(ci, b):
        return pltpu.async_copy(ones_v, deg_sh.at[dst_v.at[ci]], sems[b], add=True)

    def pipe_body(j, carry):
        for b in range(NBUF_DEG):
            ci = j * NBUF_DEG + b

            @pl.when(ci >= NBUF_DEG)
            def _wait_prev():
                pltpu.make_async_copy(
                    ones_v, deg_sh.at[dst_v.at[ci - NBUF_DEG]], sems[b]
                ).wait()

            scat(ci, b)
        return carry

    lax.fori_loop(0, CHUNKS_PER_W // NBUF_DEG, pipe_body, 0)
    for b in range(NBUF_DEG):
        ci = CHUNKS_PER_W - NBUF_DEG + b
        pltpu.make_async_copy(ones_v, deg_sh.at[dst_v.at[ci]], sems[b % NBUF_DEG]).wait()

    plsc.subcore_barrier()
    pltpu.sync_copy(
        deg_sh.at[pl.ds(row0, ROWS_PER_TILE)],
        deg_out.at[cid, pl.ds(row0, ROWS_PER_TILE)],
    )


@functools.cache
def _sc_degree_kernel():
    return pl.kernel(
        _sc_degree_body,
        out_type=jax.ShapeDtypeStruct((NC, N_PAD), jnp.float32),
        mesh=_sc_mesh(),
        scratch_types=[
            pltpu.VMEM((CHUNKS_PER_W, CHUNK), jnp.int32),
            pltpu.VMEM((CHUNK,), jnp.float32),
            pltpu.VMEM((CHUNK,), jnp.float32),
            pltpu.SemaphoreType.DMA,
            pltpu.SemaphoreType.DMA,
            pltpu.SemaphoreType.DMA,
            pltpu.SemaphoreType.DMA,
            pltpu.VMEM_SHARED((N_PAD,), jnp.float32),
        ],
    )


def _sc_degree(dst_p):
    return _sc_degree_kernel()(dst_p)


NBUF = 3


def _sc_propagate_body(idx_hbm, g_hbm, acc_out, idx_v, src_c, dst_c,
                       rows0, rows1, rows2, gs0, gs1, gs2,
                       is0, is1, is2, acc_sh):
    cid = lax.axis_index("c")
    sid = lax.axis_index("s")
    wid = _worker_id()
    bufs = (rows0, rows1, rows2)
    gsems = (gs0, gs1, gs2)
    isems = (is0, is1, is2)

    def idx_load(ci, slot):
        return pltpu.async_copy(idx_hbm.at[wid, ci], idx_v.at[slot], isems[slot])

    def idx_wait(ci, slot):
        pltpu.make_async_copy(idx_hbm.at[wid, ci], idx_v.at[slot], isems[slot]).wait()

    def decode(slot):
        for i in range(CHUNK // 16):
            w = idx_v[slot, pl.ds(i * 16, 16)]
            src_c[slot, pl.ds(i * 16, 16)] = lax.bitwise_and(w, jnp.int32(0xFFFF))
            dst_c[slot, pl.ds(i * 16, 16)] = lax.shift_right_logical(w, jnp.int32(16))

    for k in range(NBUF):
        idx_load(k, k)

    def zero_body(i, carry):
        r = i // (D // 16)
        col = (i % (D // 16)) * 16
        rows0[r, pl.ds(col, 16)] = jnp.zeros((16,), jnp.float32)
        return carry

    lax.fori_loop(0, ZBLK * (D // 16), zero_body, 0)
    row0 = sid * ROWS_PER_TILE
    for j in range(ROWS_PER_TILE // ZBLK):
        pltpu.sync_copy(
            rows0.at[pl.ds(0, ZBLK)], acc_sh.at[pl.ds(row0 + j * ZBLK, ZBLK)]
        )
    plsc.subcore_barrier()

    def gather(b):
        return pltpu.async_copy(g_hbm.at[src_c.at[b]], bufs[b], gsems[b])

    for k in range(NBUF - 1):
        idx_wait(k, k)
        decode(k)
        gather(k)

    def pipe_body(j, carry):
        for b in range(NBUF):
            ci = j * NBUF + b
            pltpu.make_async_copy(g_hbm.at[src_c.at[b]], bufs[b], gsems[b]).wait()
            sn = (b + NBUF - 1) % NBUF

            @pl.when(ci + NBUF - 1 < CHUNKS_PER_W)
            def _issue_next():
                idx_wait(ci + NBUF - 1, sn)
                decode(sn)

                @pl.when(ci + NBUF < CHUNKS_PER_W)
                def _refill_idx():
                    idx_load(ci + NBUF, b)

                gather(sn)

            pltpu.sync_copy(bufs[b], acc_sh.at[dst_c.at[b]], add=True)

        return carry

    lax.fori_loop(0, CHUNKS_PER_W // NBUF, pipe_body, 0)

    plsc.subcore_barrier()
    pltpu.sync_copy(
        acc_sh.at[pl.ds(row0, ROWS_PER_TILE)],
        acc_out.at[cid, pl.ds(row0, ROWS_PER_TILE)],
    )


@functools.cache
def _sc_propagate_kernel():
    return pl.kernel(
        _sc_propagate_body,
        out_type=jax.ShapeDtypeStruct((NC, N_PAD, D), jnp.float32),
        mesh=_sc_mesh(),
        scratch_types=[
            pltpu.VMEM((NBUF, CHUNK), jnp.int32),
            pltpu.VMEM((NBUF, CHUNK), jnp.int32),
            pltpu.VMEM((NBUF, CHUNK), jnp.int32),
            pltpu.VMEM((CHUNK, D), jnp.float32),
            pltpu.VMEM((CHUNK, D), jnp.float32),
            pltpu.VMEM((CHUNK, D), jnp.float32),
            pltpu.SemaphoreType.DMA,
            pltpu.SemaphoreType.DMA,
            pltpu.SemaphoreType.DMA,
            pltpu.SemaphoreType.DMA,
            pltpu.SemaphoreType.DMA,
            pltpu.SemaphoreType.DMA,
            pltpu.VMEM_SHARED((N_PAD, D), jnp.float32),
        ],
    )


def _sc_propagate(idx_p, g):
    return _sc_propagate_kernel()(idx_p, g)


def _tc_prep_body(deg_ref, x_ref, emb_ref, w1_ref, dis_ref, g1_ref):
    deg = deg_ref[:, 0:1] + deg_ref[:, 1:2] + 1.0
    dis = lax.rsqrt(deg)
    dis_ref[...] = dis
    oh = (x_ref[...] == lax.broadcasted_iota(jnp.int32, (BN, NUM_CLASSES_PAD), 1))
    table = jnp.dot(emb_ref[...], w1_ref[...], preferred_element_type=jnp.float32)
    g1_ref[...] = dis * jnp.dot(
        oh.astype(jnp.float32), table, preferred_element_type=jnp.float32
    )


def _tc_prep(deg_t, x2, emb32, w1):
    grid = N_NODES // BN
    return pl.pallas_call(
        _tc_prep_body,
        grid=(grid,),
        in_specs=[
            pl.BlockSpec((BN, NC), lambda i: (i, 0)),
            pl.BlockSpec((BN, 1), lambda i: (i, 0)),
            pl.BlockSpec((NUM_CLASSES_PAD, D), lambda i: (0, 0)),
            pl.BlockSpec((D, D), lambda i: (0, 0)),
        ],
        out_specs=[
            pl.BlockSpec((BN, 1), lambda i: (i, 0)),
            pl.BlockSpec((BN, D), lambda i: (i, 0)),
        ],
        out_shape=[
            jax.ShapeDtypeStruct((N_NODES, 1), jnp.float32),
            jax.ShapeDtypeStruct((N_NODES, D), jnp.float32),
        ],
    )(deg_t, x2, emb32, w1)


def _tc_mid_body(acc_ref, g_ref, dis_ref, b_ref, w_ref, out_ref):
    s = acc_ref[0] + acc_ref[1] + g_ref[...]
    h = jnp.maximum(dis_ref[...] * s + b_ref[...], 0.0)
    out_ref[...] = dis_ref[...] * jnp.dot(
        h, w_ref[...], preferred_element_type=jnp.float32
    )


def _tc_mid(acc, g, dis, b, w_next):
    grid = N_NODES // BN
    return pl.pallas_call(
        _tc_mid_body,
        grid=(grid,),
        in_specs=[
            pl.BlockSpec((NC, BN, D), lambda i: (0, i, 0)),
            pl.BlockSpec((BN, D), lambda i: (i, 0)),
            pl.BlockSpec((BN, 1), lambda i: (i, 0)),
            pl.BlockSpec((1, D), lambda i: (0, 0)),
            pl.BlockSpec((D, D), lambda i: (0, 0)),
        ],
        out_specs=pl.BlockSpec((BN, D), lambda i: (i, 0)),
        out_shape=jax.ShapeDtypeStruct((N_NODES, D), jnp.float32),
    )(acc, g, dis, b, w_next)


def _tc_final_body(acc_ref, g_ref, dis_ref, b_ref, batch_ref, out_ref, sums_ref, cnts_ref):
    i = pl.program_id(0)

    @pl.when(i == 0)
    def _init():
        sums_ref[...] = jnp.zeros_like(sums_ref)
        cnts_ref[...] = jnp.zeros_like(cnts_ref)

    s = acc_ref[0] + acc_ref[1] + g_ref[...]
    h = dis_ref[...] * s + b_ref[...]
    oh_t = (
        lax.broadcasted_iota(jnp.int32, (NUM_GRAPHS, BN), 0) == batch_ref[0]
    ).astype(jnp.float32)
    sums_ref[...] += jnp.dot(oh_t, h, preferred_element_type=jnp.float32)
    cnts_ref[...] += jnp.sum(oh_t, axis=1, keepdims=True)

    @pl.when(i == pl.num_programs(0) - 1)
    def _finish():
        pooled = sums_ref[...] / jnp.maximum(cnts_ref[...], 1.0)
        nrm = jnp.sqrt(jnp.sum(pooled * pooled, axis=1, keepdims=True))
        out_ref[...] = pooled / nrm


def _tc_final(acc, g, dis, b, batch1):
    grid = N_NODES // BN
    return pl.pallas_call(
        _tc_final_body,
        grid=(grid,),
        in_specs=[
            pl.BlockSpec((NC, BN, D), lambda i: (0, i, 0)),
            pl.BlockSpec((BN, D), lambda i: (i, 0)),
            pl.BlockSpec((BN, 1), lambda i: (i, 0)),
            pl.BlockSpec((1, D), lambda i: (0, 0)),
            pl.BlockSpec((1, 1, BN), lambda i: (i, 0, 0)),
        ],
        out_specs=pl.BlockSpec((NUM_GRAPHS, D), lambda i: (0, 0)),
        out_shape=jax.ShapeDtypeStruct((NUM_GRAPHS, D), jnp.float32),
        scratch_shapes=[
            pltpu.VMEM((NUM_GRAPHS, D), jnp.float32),
            pltpu.VMEM((NUM_GRAPHS, 1), jnp.float32),
        ],
    )(acc, g, dis, b, batch1)


def kernel(x, edge_index, batch, emb_table, W1, b1, W2, b2, W3, b3):
    src = edge_index[0]
    dst = edge_index[1]
    e = src.shape[0]
    npad = E_PAD - e
    pi = jnp.arange(npad, dtype=src.dtype)
    src_f = jnp.concatenate([src, pi % N_NODES]).astype(jnp.int32)
    dst_f = jnp.concatenate([dst, N_NODES + pi % (N_PAD - N_NODES)]).astype(jnp.int32)
    dst_p = dst_f.reshape(NW, CHUNKS_PER_W, CHUNK)
    idx_p = (src_f | (dst_f << 16)).reshape(NW, CHUNKS_PER_W, CHUNK)

    emb32 = jnp.pad(emb_table, ((0, NUM_CLASSES_PAD - emb_table.shape[0]), (0, 0)))
    x2 = x.reshape(N_NODES, 1).astype(jnp.int32)
    batch1 = batch.reshape(N_NODES // BN, 1, BN).astype(jnp.int32)

    deg2 = _sc_degree(dst_p)
    deg_t = deg2[:, :N_NODES].T
    dis, g1 = _tc_prep(deg_t, x2, emb32, W1)
    acc1 = _sc_propagate(idx_p, g1)
    g2 = _tc_mid(acc1, g1, dis, b1.reshape(1, D), W2)
    acc2 = _sc_propagate(idx_p, g2)
    g3 = _tc_mid(acc2, g2, dis, b2.reshape(1, D), W3)
    acc3 = _sc_propagate(idx_p, g3)
    return _tc_final(acc3, g3, dis, b3.reshape(1, D), batch1)


# speedup vs baseline: 1.2856x; 1.0222x over previous
import functools

import jax
import jax.numpy as jnp
from jax import lax
from jax.experimental import pallas as pl
from jax.experimental.pallas import tpu as pltpu
from jax.experimental.pallas import tpu_sc as plsc

N_NODES = 10000
N_PAD = 10240
D = 128
NUM_CLASSES_PAD = 32
NUM_GRAPHS = 64

NC = 2
NS = 16
NW = NC * NS
CHUNK = 112
CHUNKS_PER_W = 90
ZBLK = 32
E_PER_W = CHUNK * CHUNKS_PER_W
E_PAD = NW * E_PER_W
ROWS_PER_TILE = N_PAD // NS

BN = 2000

@functools.cache
def _sc_mesh():
    return plsc.VectorSubcoreMesh(
        core_axis_name="c", subcore_axis_name="s", num_cores=NC, num_subcores=NS
    )


def _worker_id():
    return lax.axis_index("s") * NC + lax.axis_index("c")


NBUF_DEG = 3


def _sc_degree_body(dst_hbm, deg_out, dst_v, ones_v, zero_v,
                    ds0, ds1, ds2, ds3, deg_sh):
    cid = lax.axis_index("c")
    sid = lax.axis_index("s")
    wid = _worker_id()
    sems = (ds0, ds1, ds2, ds3)

    pltpu.sync_copy(dst_hbm.at[wid], dst_v)

    for i in range(CHUNK // 16):
        ones_v[pl.ds(i * 16, 16)] = jnp.ones((16,), jnp.float32)
        zero_v[pl.ds(i * 16, 16)] = jnp.zeros((16,), jnp.float32)

    row0 = sid * ROWS_PER_TILE
    for j in range(ROWS_PER_TILE // ZBLK):
        pltpu.sync_copy(
            zero_v.at[pl.ds(0, ZBLK)], deg_sh.at[pl.ds(row0 + j * ZBLK, ZBLK)]
        )
    plsc.subcore_barrier()

    def scat(ci, b):
        return pltpu.async_copy(ones_v, deg_sh.at[dst_v.at[ci]], sems[b], add=True)

    def pipe_body(j, carry):
        for b in range(NBUF_DEG):
            ci = j * NBUF_DEG + b

            @pl.when(ci >= NBUF_DEG)
            def _wait_prev():
                pltpu.make_async_copy(
                    ones_v, deg_sh.at[dst_v.at[ci - NBUF_DEG]], sems[b]
                ).wait()

            scat(ci, b)
        return carry

    lax.fori_loop(0, CHUNKS_PER_W // NBUF_DEG, pipe_body, 0)
    for b in range(NBUF_DEG):
        ci = CHUNKS_PER_W - NBUF_DEG + b
        pltpu.make_async_copy(ones_v, deg_sh.at[dst_v.at[ci]], sems[b % NBUF_DEG]).wait()

    plsc.subcore_barrier()
    pltpu.sync_copy(
        deg_sh.at[pl.ds(row0, ROWS_PER_TILE)],
        deg_out.at[cid, pl.ds(row0, ROWS_PER_TILE)],
    )


@functools.cache
def _sc_degree_kernel():
    return pl.kernel(
        _sc_degree_body,
        out_type=jax.ShapeDtypeStruct((NC, N_PAD), jnp.float32),
        mesh=_sc_mesh(),
        scratch_types=[
            pltpu.VMEM((CHUNKS_PER_W, CHUNK), jnp.int32),
            pltpu.VMEM((CHUNK,), jnp.float32),
            pltpu.VMEM((CHUNK,), jnp.float32),
            pltpu.SemaphoreType.DMA,
            pltpu.SemaphoreType.DMA,
            pltpu.SemaphoreType.DMA,
            pltpu.SemaphoreType.DMA,
            pltpu.VMEM_SHARED((N_PAD,), jnp.float32),
        ],
    )


def _sc_degree(dst_p):
    return _sc_degree_kernel()(dst_p)


NBUF = 3


def _sc_propagate_body(idx_hbm, g_hbm, acc_out, idx_v, src_c, dst_c,
                       rows0, rows1, rows2, gs0, gs1, gs2,
                       is0, is1, is2, acc_sh):
    cid = lax.axis_index("c")
    sid = lax.axis_index("s")
    wid = _worker_id()
    bufs = (rows0, rows1, rows2)
    gsems = (gs0, gs1, gs2)
    isems = (is0, is1, is2)

    def idx_load(ci, slot):
        return pltpu.async_copy(idx_hbm.at[wid, ci], idx_v.at[slot], isems[slot])

    def idx_wait(ci, slot):
        pltpu.make_async_copy(idx_hbm.at[wid, ci], idx_v.at[slot], isems[slot]).wait()

    def decode(slot):
        for i in range(CHUNK // 16):
            w = idx_v[slot, pl.ds(i * 16, 16)]
            src_c[slot, pl.ds(i * 16, 16)] = lax.bitwise_and(w, jnp.int32(0xFFFF))
            dst_c[slot, pl.ds(i * 16, 16)] = lax.shift_right_logical(w, jnp.int32(16))

    for k in range(NBUF):
        idx_load(k, k)

    def zero_body(i, carry):
        r = i // (D // 16)
        col = (i % (D // 16)) * 16
        rows0[r, pl.ds(col, 16)] = jnp.zeros((16,), jnp.float32)
        return carry

    lax.fori_loop(0, ZBLK * (D // 16), zero_body, 0)
    row0 = sid * ROWS_PER_TILE
    for j in range(ROWS_PER_TILE // ZBLK):
        pltpu.sync_copy(
            rows0.at[pl.ds(0, ZBLK)], acc_sh.at[pl.ds(row0 + j * ZBLK, ZBLK)]
        )
    plsc.subcore_barrier()

    def gather(b):
        return pltpu.async_copy(g_hbm.at[src_c.at[b]], bufs[b], gsems[b])

    for k in range(NBUF - 1):
        idx_wait(k, k)
        decode(k)
        gather(k)

    def pipe_body(j, carry):
        for b in range(NBUF):
            ci = j * NBUF + b
            pltpu.make_async_copy(g_hbm.at[src_c.at[b]], bufs[b], gsems[b]).wait()
            sn = (b + NBUF - 1) % NBUF

            @pl.when(ci + NBUF - 1 < CHUNKS_PER_W)
            def _issue_next():
                idx_wait(ci + NBUF - 1, sn)
                decode(sn)

                @pl.when(ci + NBUF < CHUNKS_PER_W)
                def _refill_idx():
                    idx_load(ci + NBUF, b)

                gather(sn)

            pltpu.sync_copy(bufs[b], acc_sh.at[dst_c.at[b]], add=True)

        return carry

    lax.fori_loop(0, CHUNKS_PER_W // NBUF, pipe_body, 0)

    plsc.subcore_barrier()
    pltpu.sync_copy(
        acc_sh.at[pl.ds(row0, ROWS_PER_TILE)],
        acc_out.at[cid, pl.ds(row0, ROWS_PER_TILE)],
    )


@functools.cache
def _sc_propagate_kernel():
    return pl.kernel(
        _sc_propagate_body,
        out_type=jax.ShapeDtypeStruct((NC, N_PAD, D), jnp.float32),
        mesh=_sc_mesh(),
        scratch_types=[
            pltpu.VMEM((NBUF, CHUNK), jnp.int32),
            pltpu.VMEM((NBUF, CHUNK), jnp.int32),
            pltpu.VMEM((NBUF, CHUNK), jnp.int32),
            pltpu.VMEM((CHUNK, D), jnp.float32),
            pltpu.VMEM((CHUNK, D), jnp.float32),
            pltpu.VMEM((CHUNK, D), jnp.float32),
            pltpu.SemaphoreType.DMA,
            pltpu.SemaphoreType.DMA,
            pltpu.SemaphoreType.DMA,
            pltpu.SemaphoreType.DMA,
            pltpu.SemaphoreType.DMA,
            pltpu.SemaphoreType.DMA,
            pltpu.VMEM_SHARED((N_PAD, D), jnp.float32),
        ],
    )


def _sc_propagate(idx_p, g):
    return _sc_propagate_kernel()(idx_p, g)


def _tc_prep_body(deg_ref, x_ref, emb_ref, w1_ref, dis_ref, g1_ref):
    deg = deg_ref[:, 0:1] + deg_ref[:, 1:2] + 1.0
    dis = lax.rsqrt(deg)
    dis_ref[...] = dis
    oh = (x_ref[...] == lax.broadcasted_iota(jnp.int32, (BN, NUM_CLASSES_PAD), 1))
    table = jnp.dot(emb_ref[...], w1_ref[...], preferred_element_type=jnp.float32)
    g1_ref[...] = dis * jnp.dot(
        oh.astype(jnp.float32), table, preferred_element_type=jnp.float32
    )


def _tc_prep(deg_t, x2, emb32, w1):
    grid = N_NODES // BN
    return pl.pallas_call(
        _tc_prep_body,
        grid=(grid,),
        in_specs=[
            pl.BlockSpec((BN, NC), lambda i: (i, 0)),
            pl.BlockSpec((BN, 1), lambda i: (i, 0)),
            pl.BlockSpec((NUM_CLASSES_PAD, D), lambda i: (0, 0)),
            pl.BlockSpec((D, D), lambda i: (0, 0)),
        ],
        out_specs=[
            pl.BlockSpec((BN, 1), lambda i: (i, 0)),
            pl.BlockSpec((BN, D), lambda i: (i, 0)),
        ],
        out_shape=[
            jax.ShapeDtypeStruct((N_NODES, 1), jnp.float32),
            jax.ShapeDtypeStruct((N_NODES, D), jnp.float32),
        ],
    )(deg_t, x2, emb32, w1)


def _tc_mid_body(acc_ref, g_ref, dis_ref, b_ref, w_ref, out_ref):
    s = acc_ref[0] + acc_ref[1] + g_ref[...]
    h = jnp.maximum(dis_ref[...] * s + b_ref[...], 0.0)
    out_ref[...] = dis_ref[...] * jnp.dot(
        h, w_ref[...], preferred_element_type=jnp.float32
    )


def _tc_mid(acc, g, dis, b, w_next):
    grid = N_NODES // BN
    return pl.pallas_call(
        _tc_mid_body,
        grid=(grid,),
        in_specs=[
            pl.BlockSpec((NC, BN, D), lambda i: (0, i, 0)),
            pl.BlockSpec((BN, D), lambda i: (i, 0)),
            pl.BlockSpec((BN, 1), lambda i: (i, 0)),
            pl.BlockSpec((1, D), lambda i: (0, 0)),
            pl.BlockSpec((D, D), lambda i: (0, 0)),
        ],
        out_specs=pl.BlockSpec((BN, D), lambda i: (i, 0)),
        out_shape=jax.ShapeDtypeStruct((N_NODES, D), jnp.float32),
    )(acc, g, dis, b, w_next)


def _tc_final_body(acc_ref, g_ref, dis_ref, b_ref, batch_ref, out_ref, sums_ref, cnts_ref):
    i = pl.program_id(0)

    @pl.when(i == 0)
    def _init():
        sums_ref[...] = jnp.zeros_like(sums_ref)
        cnts_ref[...] = jnp.zeros_like(cnts_ref)

    s = acc_ref[0] + acc_ref[1] + g_ref[...]
    h = dis_ref[...] * s + b_ref[...]
    oh_t = (
        lax.broadcasted_iota(jnp.int32, (NUM_GRAPHS, BN), 0) == batch_ref[0]
    ).astype(jnp.float32)
    sums_ref[...] += jnp.dot(oh_t, h, preferred_element_type=jnp.float32)
    cnts_ref[...] += jnp.sum(oh_t, axis=1, keepdims=True)

    @pl.when(i == pl.num_programs(0) - 1)
    def _finish():
        pooled = sums_ref[...] / jnp.maximum(cnts_ref[...], 1.0)
        nrm = jnp.sqrt(jnp.sum(pooled * pooled, axis=1, keepdims=True))
        out_ref[...] = pooled / nrm


def _tc_final(acc, g, dis, b, batch1):
    grid = N_NODES // BN
    return pl.pallas_call(
        _tc_final_body,
        grid=(grid,),
        in_specs=[
            pl.BlockSpec((NC, BN, D), lambda i: (0, i, 0)),
            pl.BlockSpec((BN, D), lambda i: (i, 0)),
            pl.BlockSpec((BN, 1), lambda i: (i, 0)),
            pl.BlockSpec((1, D), lambda i: (0, 0)),
            pl.BlockSpec((1, 1, BN), lambda i: (i, 0, 0)),
        ],
        out_specs=pl.BlockSpec((NUM_GRAPHS, D), lambda i: (0, 0)),
        out_shape=jax.ShapeDtypeStruct((NUM_GRAPHS, D), jnp.float32),
        scratch_shapes=[
            pltpu.VMEM((NUM_GRAPHS, D), jnp.float32),
            pltpu.VMEM((NUM_GRAPHS, 1), jnp.float32),
        ],
    )(acc, g, dis, b, batch1)


def kernel(x, edge_index, batch, emb_table, W1, b1, W2, b2, W3, b3):
    src = edge_index[0]
    dst = edge_index[1]
    e = src.shape[0]
    npad = E_PAD - e
    pi = jnp.arange(npad, dtype=src.dtype)
    src_f = jnp.concatenate([src, pi % N_NODES]).astype(jnp.int32)
    dst_f = jnp.concatenate([dst, N_NODES + pi % (N_PAD - N_NODES)]).astype(jnp.int32)
    dst_p = dst_f.reshape(NW, CHUNKS_PER_W, CHUNK)
    idx_p = (src_f | (dst_f << 16)).reshape(NW, CHUNKS_PER_W, CHUNK)

    emb32 = jnp.pad(emb_table, ((0, NUM_CLASSES_PAD - emb_table.shape[0]), (0, 0)))
    x2 = x.reshape(N_NODES, 1).astype(jnp.int32)
    batch1 = batch.reshape(N_NODES // BN, 1, BN).astype(jnp.int32)

    deg2 = _sc_degree(dst_p)
    deg_t = deg2[:, :N_NODES].T
    dis, g1 = _tc_prep(deg_t, x2, emb32, W1)
    acc1 = _sc_propagate(idx_p, g1)
    g2 = _tc_mid(acc1, g1, dis, b1.reshape(1, D), W2)
    acc2 = _sc_propagate(idx_p, g2)
    g3 = _tc_mid(acc2, g2, dis, b2.reshape(1, D), W3)
    acc3 = _sc_propagate(idx_p, g3)
    return _tc_final(acc3, g3, dis, b3.reshape(1, D), batch1)
